# baseline (speedup 1.0000x reference)
import jax
import jax.numpy as jnp
from jax.experimental import pallas as pl

_THETA_ALPHA = 16.0
_THETA_BETA = 0.5
_T = 1024
_D_PAD = 16
_C_PAD = 32


def _pairwise_body(at_ref, b_ref, xt_ref, o_ref):
    n = b_ref.shape[0]
    nb = n // _T
    o_ref[...] = jnp.zeros_like(o_ref)

    def tile_k(i, j):
        ai = at_ref[pl.ds(i * _T, _T), :]
        bj = b_ref[pl.ds(j * _T, _T), :]
        e = jax.lax.dot_general(
            bj, ai, (((1,), (1,)), ((), ())),
            preferred_element_type=jnp.float32)
        return jnp.exp(jnp.minimum(e, 0.0)).astype(jnp.bfloat16)

    def accum(i, j, k):
        xtj = xt_ref[:, pl.ds(j * _T, _T)]
        o_ref[:, pl.ds(i * _T, _T)] += jax.lax.dot_general(
            xtj, k, (((1,), (0,)), ((), ())),
            preferred_element_type=jnp.float32)
        xti = xt_ref[:, pl.ds(i * _T, _T)]
        o_ref[:, pl.ds(j * _T, _T)] += jax.lax.dot_general(
            xti, k, (((1,), (1,)), ((), ())),
            preferred_element_type=jnp.float32)

    def pair2(i1, j1, i2, j2):
        k1 = tile_k(i1, j1)
        k2 = tile_k(i2, j2)
        accum(i1, j1, k1)
        accum(i2, j2, k2)

    def diag2(t, carry):
        i1, i2 = 2 * t, 2 * t + 1
        k1 = tile_k(i1, i1)
        k2 = tile_k(i2, i2)
        xt1 = xt_ref[:, pl.ds(i1 * _T, _T)]
        o_ref[:, pl.ds(i1 * _T, _T)] += jax.lax.dot_general(
            xt1, k1, (((1,), (0,)), ((), ())),
            preferred_element_type=jnp.float32)
        xt2 = xt_ref[:, pl.ds(i2 * _T, _T)]
        o_ref[:, pl.ds(i2 * _T, _T)] += jax.lax.dot_general(
            xt2, k2, (((1,), (0,)), ((), ())),
            preferred_element_type=jnp.float32)
        return carry

    jax.lax.fori_loop(0, nb // 2, diag2, 0)

    for d in range(1, nb // 2):
        def offd(t, carry, d=d):
            i1, i2 = 2 * t, 2 * t + 1
            pair2(i1, (i1 + d) % nb, i2, (i2 + d) % nb)
            return carry

        jax.lax.fori_loop(0, nb // 2, offd, 0)

    def half(t, carry):
        i1, i2 = 2 * t, 2 * t + 1
        pair2(i1, i1 + nb // 2, i2, i2 + nb // 2)
        return carry

    jax.lax.fori_loop(0, nb // 4, half, 0)


def _gauss_filter_pallas(x_flat, f):
    n, c = x_flat.shape
    d = f.shape[1]
    fb = f.astype(jnp.bfloat16)
    hc = -0.5 * jnp.sum(f * f, axis=-1, keepdims=True)
    h_hi = jax.lax.optimization_barrier(hc.astype(jnp.bfloat16))
    h_lo = (hc - h_hi.astype(jnp.float32)).astype(jnp.bfloat16)
    ones = jnp.ones((n, 1), jnp.bfloat16)
    pad = jnp.zeros((n, _D_PAD - d - 4), jnp.bfloat16)
    at = jnp.concatenate([fb, h_hi, h_lo, ones, ones, pad], axis=1)
    bb = jnp.concatenate([fb, ones, ones, h_hi, h_lo, pad], axis=1)
    xt = jnp.zeros((_C_PAD, n), jnp.bfloat16).at[:c, :].set(
        x_flat.T.astype(jnp.bfloat16))

    out_t = pl.pallas_call(
        _pairwise_body,
        grid=(1,),
        in_specs=[
            pl.BlockSpec((n, _D_PAD), lambda i: (0, 0)),
            pl.BlockSpec((n, _D_PAD), lambda i: (0, 0)),
            pl.BlockSpec((_C_PAD, n), lambda i: (0, 0)),
        ],
        out_specs=pl.BlockSpec((_C_PAD, n), lambda i: (0, 0)),
        out_shape=jax.ShapeDtypeStruct((_C_PAD, n), jnp.float32),
    )(at, bb, xt)
    return out_t[:c, :]


def kernel(x, image):
    bsz, c, h, w = x.shape
    n = h * w
    yy, xx = jnp.meshgrid(
        jnp.arange(h, dtype=jnp.float32),
        jnp.arange(w, dtype=jnp.float32),
        indexing="ij",
    )
    pos = jnp.stack([yy, xx], axis=-1).reshape(n, 2) / _THETA_ALPHA

    outs = []
    for bi in range(bsz):
        img_flat = image[bi].reshape(image.shape[1], n).T / _THETA_BETA
        f = jnp.concatenate([pos, img_flat], axis=1)
        x_flat = x[bi].reshape(c, n).T
        out_t = _gauss_filter_pallas(x_flat, f)
        outs.append(out_t.reshape(c, h, w))
    return jnp.stack(outs, axis=0)


# speedup vs baseline: 1.0052x; 1.0052x over previous
import jax
import jax.numpy as jnp
from jax.experimental import pallas as pl

_THETA_ALPHA = 16.0
_THETA_BETA = 0.5
_T = 1024
_D_PAD = 16
_C_PAD = 32


def _pairwise_body(at_ref, b_ref, xt_ref, o_ref):
    n = b_ref.shape[0]
    nb = n // _T
    o_ref[...] = jnp.zeros_like(o_ref)

    def tile_k(i, j):
        ati = at_ref[:, pl.ds(i * _T, _T)]
        bj = b_ref[pl.ds(j * _T, _T), :]
        e = jax.lax.dot_general(
            bj, ati, (((1,), (0,)), ((), ())),
            preferred_element_type=jnp.float32)
        return jnp.exp(jnp.minimum(e, 0.0)).astype(jnp.bfloat16)

    def accum(i, j, k):
        xtj = xt_ref[:, pl.ds(j * _T, _T)]
        o_ref[:, pl.ds(i * _T, _T)] += jax.lax.dot_general(
            xtj, k, (((1,), (0,)), ((), ())),
            preferred_element_type=jnp.float32)
        xti = xt_ref[:, pl.ds(i * _T, _T)]
        o_ref[:, pl.ds(j * _T, _T)] += jax.lax.dot_general(
            xti, k, (((1,), (1,)), ((), ())),
            preferred_element_type=jnp.float32)

    def pair2(i1, j1, i2, j2):
        k1 = tile_k(i1, j1)
        k2 = tile_k(i2, j2)
        accum(i1, j1, k1)
        accum(i2, j2, k2)

    def diag2(t, carry):
        i1, i2 = 2 * t, 2 * t + 1
        k1 = tile_k(i1, i1)
        k2 = tile_k(i2, i2)
        xt1 = xt_ref[:, pl.ds(i1 * _T, _T)]
        o_ref[:, pl.ds(i1 * _T, _T)] += jax.lax.dot_general(
            xt1, k1, (((1,), (0,)), ((), ())),
            preferred_element_type=jnp.float32)
        xt2 = xt_ref[:, pl.ds(i2 * _T, _T)]
        o_ref[:, pl.ds(i2 * _T, _T)] += jax.lax.dot_general(
            xt2, k2, (((1,), (0,)), ((), ())),
            preferred_element_type=jnp.float32)
        return carry

    jax.lax.fori_loop(0, nb // 2, diag2, 0)

    for d in range(1, nb // 2):
        def offd(t, carry, d=d):
            i1, i2 = 2 * t, 2 * t + 1
            pair2(i1, (i1 + d) % nb, i2, (i2 + d) % nb)
            return carry

        jax.lax.fori_loop(0, nb // 2, offd, 0)

    def half(t, carry):
        i1, i2 = 2 * t, 2 * t + 1
        pair2(i1, i1 + nb // 2, i2, i2 + nb // 2)
        return carry

    jax.lax.fori_loop(0, nb // 4, half, 0)


def _gauss_filter_pallas(x_flat, f):
    n, c = x_flat.shape
    d = f.shape[1]
    fb = f.astype(jnp.bfloat16)
    hc = -0.5 * jnp.sum(f * f, axis=-1, keepdims=True)
    h_hi = jax.lax.optimization_barrier(hc.astype(jnp.bfloat16))
    h_lo = (hc - h_hi.astype(jnp.float32)).astype(jnp.bfloat16)
    ones = jnp.ones((n, 1), jnp.bfloat16)
    pad = jnp.zeros((n, _D_PAD - d - 4), jnp.bfloat16)
    at = jnp.concatenate([fb, h_hi, h_lo, ones, ones, pad], axis=1).T
    bb = jnp.concatenate([fb, ones, ones, h_hi, h_lo, pad], axis=1)
    xt = jnp.zeros((_C_PAD, n), jnp.bfloat16).at[:c, :].set(
        x_flat.T.astype(jnp.bfloat16))

    out_t = pl.pallas_call(
        _pairwise_body,
        grid=(1,),
        in_specs=[
            pl.BlockSpec((_D_PAD, n), lambda i: (0, 0)),
            pl.BlockSpec((n, _D_PAD), lambda i: (0, 0)),
            pl.BlockSpec((_C_PAD, n), lambda i: (0, 0)),
        ],
        out_specs=pl.BlockSpec((_C_PAD, n), lambda i: (0, 0)),
        out_shape=jax.ShapeDtypeStruct((_C_PAD, n), jnp.float32),
    )(at, bb, xt)
    return out_t[:c, :]


def kernel(x, image):
    bsz, c, h, w = x.shape
    n = h * w
    yy, xx = jnp.meshgrid(
        jnp.arange(h, dtype=jnp.float32),
        jnp.arange(w, dtype=jnp.float32),
        indexing="ij",
    )
    pos = jnp.stack([yy, xx], axis=-1).reshape(n, 2) / _THETA_ALPHA

    outs = []
    for bi in range(bsz):
        img_flat = image[bi].reshape(image.shape[1], n).T / _THETA_BETA
        f = jnp.concatenate([pos, img_flat], axis=1)
        x_flat = x[bi].reshape(c, n).T
        out_t = _gauss_filter_pallas(x_flat, f)
        outs.append(out_t.reshape(c, h, w))
    return jnp.stack(outs, axis=0)
